# baseline (speedup 1.0000x reference)
import functools

import jax
import jax.numpy as jnp
from jax import lax
from jax.experimental import pallas as pl
from jax.experimental.pallas import tpu as pltpu
from jax.experimental.pallas import tpu_sc as plsc

NC = 2
NS = 16
L = 16
NW = NC * NS


@functools.cache
def _build(B: int):
    BPW = B // NW
    CH = BPW // L
    TB = BPW // 128
    NB = B // 128

    mesh = plsc.VectorSubcoreMesh(core_axis_name="c", subcore_axis_name="s",
                                  num_cores=NC, num_subcores=NS)

    @functools.partial(
        pl.kernel,
        out_type=jax.ShapeDtypeStruct((NB, 8, 128), jnp.float32),
        mesh=mesh,
        scratch_types=[
            pltpu.VMEM((TB, 4, 128), jnp.float32),
            pltpu.VMEM((TB, 4, 128), jnp.int32),
            pltpu.VMEM((TB, 8, 128), jnp.float32),
            pltpu.VMEM((48,), jnp.float32),
            pltpu.VMEM((48,), jnp.float32),
            pltpu.VMEM((16,), jnp.float32),
            pltpu.VMEM((6, 64), jnp.float32),
        ],
        compiler_params=pltpu.CompilerParams(
            needs_layout_passes=False, use_tc_tiling_on_sc=False,
            skip_device_barrier=True),
    )
    def sc_encoder(cont_hbm, idx_hbm, emb_hbm, w_hbm, b_hbm, out_hbm,
                   cont_v, idx_v, out_v, emb_v, w_v, b_v, tab_v):
        wid = lax.axis_index("s") * NC + lax.axis_index("c")

        pltpu.sync_copy(cont_hbm.at[pl.ds(wid * TB, TB)], cont_v)
        pltpu.sync_copy(idx_hbm.at[pl.ds(wid * TB, TB)], idx_v)
        pltpu.sync_copy(emb_hbm, emb_v.at[pl.ds(0, 40)])
        pltpu.sync_copy(w_hbm, w_v)
        pltpu.sync_copy(b_hbm, b_v.at[pl.ds(0, 6)])

        lane = lax.iota(jnp.int32, 16)
        wch = [w_v[pl.ds(16 * r, 16)] for r in range(3)]
        bvec = b_v[...]

        def wscal(j, c):
            k = 8 * j + c
            return wch[k // 16][k % 16]

        ev = [plsc.load_gather(emb_v, [lane + 10 * i]) for i in range(4)]
        for j in range(6):
            bj = jnp.full((16,), bvec[j], jnp.float32)
            for i in range(4):
                val = ev[i] * wscal(j, 4 + i)
                if i == 0:
                    val = val + bj
                tab_v[j, pl.ds(16 * i, 16)] = val

        wd = [[wscal(j, c) for c in range(4)] for j in range(6)]
        jcol = [jnp.full((16,), j, jnp.int32) for j in range(6)]

        for t in range(CH):
            blk, cc0 = t // 8, (16 * t) % 128
            cc = [cont_v[blk, c, pl.ds(cc0, 16)] for c in range(4)]
            fidx = [idx_v[blk, i, pl.ds(cc0, 16)] + 16 * i
                    for i in range(4)]
            for j in range(6):
                acc = (cc[0] * wd[j][0] + cc[1] * wd[j][1]
                       + cc[2] * wd[j][2] + cc[3] * wd[j][3])
                acc = acc + ((plsc.load_gather(tab_v, [jcol[j], fidx[0]])
                              + plsc.load_gather(tab_v, [jcol[j], fidx[1]]))
                             + (plsc.load_gather(tab_v, [jcol[j], fidx[2]])
                                + plsc.load_gather(tab_v, [jcol[j], fidx[3]])))
                out_v[blk, j, pl.ds(cc0, 16)] = acc

        pltpu.sync_copy(out_v, out_hbm.at[pl.ds(wid * TB, TB)])

    return sc_encoder


def kernel(cont, cat_idx, emb, W, b):
    B = cont.shape[0]
    idx32 = cat_idx.astype(jnp.int32)
    cont3 = cont.reshape(B // 128, 128, 4).transpose(0, 2, 1)
    idx3 = idx32.reshape(B // 128, 128, 4).transpose(0, 2, 1)
    out3 = _build(B)(cont3, idx3, emb.reshape(40), W.reshape(48), b)
    return out3.transpose(0, 2, 1).reshape(B, 8)[:, :6]


# speedup vs baseline: 1.0655x; 1.0655x over previous
import functools

import jax
import jax.numpy as jnp
from jax import lax
from jax.experimental import pallas as pl
from jax.experimental.pallas import tpu as pltpu
from jax.experimental.pallas import tpu_sc as plsc

NC = 2
NS = 16
L = 16
NW = NC * NS


@functools.cache
def _build(B: int):
    BPW = B // NW
    CH = BPW // L
    TB = BPW // 128
    NB = B // 128

    mesh = plsc.VectorSubcoreMesh(core_axis_name="c", subcore_axis_name="s",
                                  num_cores=NC, num_subcores=NS)

    @functools.partial(
        pl.kernel,
        out_type=jax.ShapeDtypeStruct((NB, 8, 128), jnp.float32),
        mesh=mesh,
        scratch_types=[
            pltpu.VMEM((TB, 4, 128), jnp.float32),
            pltpu.VMEM((TB, 4, 128), jnp.int32),
            pltpu.VMEM((TB, 8, 128), jnp.float32),
            pltpu.VMEM((48,), jnp.float32),
            pltpu.VMEM((48,), jnp.float32),
            pltpu.VMEM((16,), jnp.float32),
            pltpu.VMEM((12, 160), jnp.float32),
        ],
        compiler_params=pltpu.CompilerParams(
            needs_layout_passes=False, use_tc_tiling_on_sc=False),
    )
    def sc_encoder(cont_hbm, idx_hbm, emb_hbm, w_hbm, b_hbm, out_hbm,
                   cont_v, idx_v, out_v, emb_v, w_v, b_v, tab_v):
        wid = lax.axis_index("s") * NC + lax.axis_index("c")

        pltpu.sync_copy(cont_hbm.at[pl.ds(wid * TB, TB)], cont_v)
        pltpu.sync_copy(idx_hbm.at[pl.ds(wid * TB, TB)], idx_v)
        pltpu.sync_copy(emb_hbm, emb_v.at[pl.ds(0, 40)])
        pltpu.sync_copy(w_hbm, w_v)
        pltpu.sync_copy(b_hbm, b_v.at[pl.ds(0, 6)])

        lane = lax.iota(jnp.int32, 16)
        wch = [w_v[pl.ds(16 * r, 16)] for r in range(3)]
        bvec = b_v[...]

        def wscal(j, c):
            k = 8 * j + c
            return wch[k // 16][k % 16]

        ev = [plsc.load_gather(emb_v, [lane + 10 * i]) for i in range(4)]
        for j in range(6):
            bj = jnp.full((16,), bvec[j], jnp.float32)
            hi0 = ev[0] * wscal(j, 4) + bj
            lo0 = ev[1] * wscal(j, 5)
            hi1 = ev[2] * wscal(j, 6)
            lo1 = ev[3] * wscal(j, 7)
            for q in range(10):
                tab_v[j, pl.ds(16 * q, 16)] = (
                    jnp.full((16,), hi0[q], jnp.float32) + lo0)
                tab_v[6 + j, pl.ds(16 * q, 16)] = (
                    jnp.full((16,), hi1[q], jnp.float32) + lo1)

        wd = [[wscal(j, c) for c in range(4)] for j in range(6)]
        jrow = [jnp.full((16,), r, jnp.int32) for r in range(12)]

        for t in range(CH):
            blk, cc0 = t // 8, (16 * t) % 128
            cc = [cont_v[blk, c, pl.ds(cc0, 16)] for c in range(4)]
            iv = [idx_v[blk, i, pl.ds(cc0, 16)] for i in range(4)]
            p0 = iv[0] * 16 + iv[1]
            p1 = iv[2] * 16 + iv[3]
            for j in range(6):
                acc = ((cc[0] * wd[j][0] + cc[1] * wd[j][1])
                       + (cc[2] * wd[j][2] + cc[3] * wd[j][3]))
                acc = acc + (plsc.load_gather(tab_v, [jrow[j], p0])
                             + plsc.load_gather(tab_v, [jrow[6 + j], p1]))
                out_v[blk, j, pl.ds(cc0, 16)] = acc

        pltpu.sync_copy(out_v, out_hbm.at[pl.ds(wid * TB, TB)])

    return sc_encoder


def kernel(cont, cat_idx, emb, W, b):
    B = cont.shape[0]
    idx32 = cat_idx.astype(jnp.int32)
    cont3 = cont.reshape(B // 128, 128, 4).transpose(0, 2, 1)
    idx3 = idx32.reshape(B // 128, 128, 4).transpose(0, 2, 1)
    out3 = _build(B)(cont3, idx3, emb.reshape(40), W.reshape(48), b)
    return out3.transpose(0, 2, 1).reshape(B, 8)[:, :6]


# speedup vs baseline: 1.0980x; 1.0305x over previous
import functools

import jax
import jax.numpy as jnp
from jax import lax
from jax.experimental import pallas as pl
from jax.experimental.pallas import tpu as pltpu
from jax.experimental.pallas import tpu_sc as plsc

NC = 2
NS = 16
L = 16
NW = NC * NS


@functools.cache
def _build(B: int):
    BPW = B // NW
    CH = BPW // L
    TB = BPW // 128
    NB = B // 128

    mesh = plsc.VectorSubcoreMesh(core_axis_name="c", subcore_axis_name="s",
                                  num_cores=NC, num_subcores=NS)

    @functools.partial(
        pl.kernel,
        out_type=jax.ShapeDtypeStruct((NB, 8, 128), jnp.float32),
        mesh=mesh,
        scratch_types=[
            pltpu.VMEM((TB, 4, 128), jnp.float32),
            pltpu.VMEM((TB, 4, 128), jnp.int32),
            pltpu.VMEM((TB, 8, 128), jnp.float32),
            pltpu.VMEM((48,), jnp.float32),
            pltpu.VMEM((48,), jnp.float32),
            pltpu.VMEM((16,), jnp.float32),
            pltpu.VMEM((12, 160), jnp.float32),
            pltpu.SemaphoreType.DMA,
        ],
        compiler_params=pltpu.CompilerParams(
            needs_layout_passes=False, use_tc_tiling_on_sc=False),
    )
    def sc_encoder(cont_hbm, idx_hbm, emb_hbm, w_hbm, b_hbm, out_hbm,
                   cont_v, idx_v, out_v, emb_v, w_v, b_v, tab_v, sem):
        wid = lax.axis_index("s") * NC + lax.axis_index("c")

        cp_cont = pltpu.async_copy(cont_hbm.at[pl.ds(wid * TB, TB)], cont_v, sem)
        cp_idx = pltpu.async_copy(idx_hbm.at[pl.ds(wid * TB, TB)], idx_v, sem)
        pltpu.sync_copy(emb_hbm, emb_v.at[pl.ds(0, 40)])
        pltpu.sync_copy(w_hbm, w_v)
        pltpu.sync_copy(b_hbm, b_v.at[pl.ds(0, 6)])

        lane = lax.iota(jnp.int32, 16)
        wch = [w_v[pl.ds(16 * r, 16)] for r in range(3)]
        bvec = b_v[...]

        def wscal(j, c):
            k = 8 * j + c
            return wch[k // 16][k % 16]

        ev = [plsc.load_gather(emb_v, [lane + 10 * i]) for i in range(4)]
        for j in range(6):
            bj = jnp.full((16,), bvec[j], jnp.float32)
            hi0 = ev[0] * wscal(j, 4) + bj
            lo0 = ev[1] * wscal(j, 5)
            hi1 = ev[2] * wscal(j, 6)
            lo1 = ev[3] * wscal(j, 7)
            for q in range(10):
                tab_v[j, pl.ds(16 * q, 16)] = (
                    jnp.full((16,), hi0[q], jnp.float32) + lo0)
                tab_v[6 + j, pl.ds(16 * q, 16)] = (
                    jnp.full((16,), hi1[q], jnp.float32) + lo1)

        wd = [[wscal(j, c) for c in range(4)] for j in range(6)]
        jrow = [jnp.full((16,), r, jnp.int32) for r in range(12)]

        cp_cont.wait()
        cp_idx.wait()

        for t in range(CH):
            blk, cc0 = t // 8, (16 * t) % 128
            cc = [cont_v[blk, c, pl.ds(cc0, 16)] for c in range(4)]
            iv = [idx_v[blk, i, pl.ds(cc0, 16)] for i in range(4)]
            p0 = iv[0] * 16 + iv[1]
            p1 = iv[2] * 16 + iv[3]
            for j in range(6):
                acc = ((cc[0] * wd[j][0] + cc[1] * wd[j][1])
                       + (cc[2] * wd[j][2] + cc[3] * wd[j][3]))
                acc = acc + (plsc.load_gather(tab_v, [jrow[j], p0])
                             + plsc.load_gather(tab_v, [jrow[6 + j], p1]))
                out_v[blk, j, pl.ds(cc0, 16)] = acc

        pltpu.sync_copy(out_v, out_hbm.at[pl.ds(wid * TB, TB)])

    return sc_encoder


def kernel(cont, cat_idx, emb, W, b):
    B = cont.shape[0]
    idx32 = cat_idx.astype(jnp.int32)
    cont3 = cont.reshape(B // 128, 128, 4).transpose(0, 2, 1)
    idx3 = idx32.reshape(B // 128, 128, 4).transpose(0, 2, 1)
    out3 = _build(B)(cont3, idx3, emb.reshape(40), W.reshape(48), b)
    return out3.transpose(0, 2, 1).reshape(B, 8)[:, :6]
